# baseline (speedup 1.0000x reference)
import functools

import jax
import jax.numpy as jnp
from jax import lax
from jax.experimental import pallas as pl
from jax.experimental.pallas import tpu as pltpu
from jax.experimental.pallas import tpu_sc as plsc

_B = 2
_CLIP = 2
_C = 144
_H = 48
_W = 48
_AREA = _H * _W
_G = 12
_CG = _C // _G
_K2 = 9
_NS = _CLIP * _K2
_BG = _B * _G
_NCH = 18
_CHUNK = _AREA // _NCH
_NBLK = _CHUNK // 16
_SCALE = float(_CG) ** -0.5

_mesh = plsc.VectorSubcoreMesh(core_axis_name="c", subcore_axis_name="s")


def _const16(v):
    return jnp.full((16,), v, jnp.int32)


@functools.partial(
    pl.kernel,
    out_type=jax.ShapeDtypeStruct((_B, _G, _CG, _AREA), jnp.float32),
    mesh=_mesh,
    scratch_types=[
        pltpu.VMEM((4 * _CG * _AREA,), jnp.float32),
        pltpu.VMEM((_CLIP, _K2, 2, _CHUNK), jnp.float32),
        pltpu.VMEM((_K2, 2, _CHUNK), jnp.float32),
        pltpu.VMEM((_CG, _CHUNK), jnp.float32),
        pltpu.VMEM((_CG, _CHUNK), jnp.float32),
        pltpu.VMEM((_NS * 4 * 16,), jnp.float32),
        pltpu.VMEM((_NS * 4 * 16,), jnp.int32),
        pltpu.VMEM((_NS * 16,), jnp.float32),
    ],
    compiler_params=pltpu.CompilerParams(needs_layout_passes=False),
)
def _sc_attn(kp_hbm, vp_hbm, qp_hbm, off_hbm, grid_hbm, out_hbm,
             kvt, offv, gridv, qv, outv, wbuf, ibuf, lbuf):
    wid = lax.axis_index("s") * 2 + lax.axis_index("c")
    ncg = _BG * _NCH
    start = (wid * ncg) // 32
    end = ((wid + 1) * ncg) // 32

    def chunk_body(cg, prev_bg):
        bg = cg // _NCH
        ch = cg - bg * _NCH
        b = bg // _G
        g = bg - b * _G

        clw = _CG * _AREA
        @pl.when(bg != prev_bg)
        def _load_table():
            for clip in range(_CLIP):
                base = (b * 2 + clip) * (_C * _AREA) + g * clw
                pltpu.sync_copy(
                    kp_hbm.at[pl.ds(base, clw)], kvt.at[pl.ds(clip * clw, clw)]
                )
                pltpu.sync_copy(
                    vp_hbm.at[pl.ds(base, clw)],
                    kvt.at[pl.ds((2 + clip) * clw, clw)],
                )

        for clip in range(_CLIP):
            pltpu.sync_copy(
                off_hbm.at[b, clip, g, :, :, pl.ds(ch * _CHUNK, _CHUNK)],
                offv.at[clip],
            )
        pltpu.sync_copy(grid_hbm.at[:, :, pl.ds(ch * _CHUNK, _CHUNK)], gridv)
        pltpu.sync_copy(qp_hbm.at[b, g, :, pl.ds(ch * _CHUNK, _CHUNK)], qv)

        def blk_body(blk, carry1):
            p0 = blk * 16
            qs = [qv[c, pl.ds(p0, 16)] for c in range(_CG)]

            def make_s1(clip):
                def s1(kpos, carry2):
                    s = kpos + clip * _K2
                    sy = offv[clip, kpos, 0, pl.ds(p0, 16)] + gridv[kpos, 0, pl.ds(p0, 16)]
                    sx = offv[clip, kpos, 1, pl.ds(p0, 16)] + gridv[kpos, 1, pl.ds(p0, 16)]
                    ty = sy.astype(jnp.int32)
                    y0 = jnp.where(sy < ty.astype(jnp.float32), ty - 1, ty)
                    tx = sx.astype(jnp.int32)
                    x0 = jnp.where(sx < tx.astype(jnp.float32), tx - 1, tx)
                    gs = []
                    ws = []
                    for t, (dy, dx) in enumerate(((0, 0), (0, 1), (1, 0), (1, 1))):
                        yi = y0 + dy
                        xi = x0 + dx
                        wy = 1.0 - jnp.abs(sy - yi.astype(jnp.float32))
                        wx = 1.0 - jnp.abs(sx - xi.astype(jnp.float32))
                        ok = (yi >= 0) & (yi <= _H - 1) & (xi >= 0) & (xi <= _W - 1)
                        wgt = jnp.where(ok, wy * wx, 0.0)
                        yc = jnp.clip(yi, 0, _H - 1)
                        xc = jnp.clip(xi, 0, _W - 1)
                        gi = yc * _W + xc
                        wbuf[pl.ds((s * 4 + t) * 16, 16)] = wgt
                        ibuf[pl.ds((s * 4 + t) * 16, 16)] = gi
                        gs.append(gi)
                        ws.append(wgt)
                    logit = jnp.zeros((16,), jnp.float32)
                    for c in range(_CG):
                        o = clip * (_CG * _AREA) + c * _AREA
                        kc = ws[0] * plsc.load_gather(kvt, [gs[0] + o])
                        for t in range(1, 4):
                            kc = kc + ws[t] * plsc.load_gather(kvt, [gs[t] + o])
                        logit = logit + qs[c] * kc
                    lbuf[pl.ds(s * 16, 16)] = logit * _SCALE
                    return carry2

                return s1

            lax.fori_loop(0, _K2, make_s1(0), 0)
            lax.fori_loop(0, _K2, make_s1(1), 0)

            def smax(s, m):
                return jnp.maximum(m, lbuf[pl.ds(s * 16, 16)])

            m = lax.fori_loop(1, _NS, smax, lbuf[pl.ds(0, 16)])

            def sexp(s, den):
                p = jnp.exp(lbuf[pl.ds(s * 16, 16)] - m)
                lbuf[pl.ds(s * 16, 16)] = p
                return den + p

            den = lax.fori_loop(0, _NS, sexp, jnp.zeros((16,), jnp.float32))
            rden = 1.0 / den

            def make_s2(clip):
                def s2(kpos, acc):
                    s = kpos + clip * _K2
                    a = lbuf[pl.ds(s * 16, 16)] * rden
                    accl = list(acc)
                    for t in range(4):
                        aw = a * wbuf[pl.ds((s * 4 + t) * 16, 16)]
                        gi = ibuf[pl.ds((s * 4 + t) * 16, 16)]
                        for c in range(_CG):
                            o = (2 + clip) * (_CG * _AREA) + c * _AREA
                            accl[c] = accl[c] + aw * plsc.load_gather(
                                kvt, [gi + o]
                            )
                    return tuple(accl)

                return s2

            zero = jnp.zeros((16,), jnp.float32)
            acc = lax.fori_loop(0, _K2, make_s2(0), (zero,) * _CG)
            acc = lax.fori_loop(0, _K2, make_s2(1), acc)
            for c in range(_CG):
                outv[c, pl.ds(p0, 16)] = acc[c]
            return carry1

        lax.fori_loop(0, _NBLK, blk_body, 0)
        pltpu.sync_copy(
            outv, out_hbm.at[b, g, :, pl.ds(ch * _CHUNK, _CHUNK)]
        )
        return bg

    lax.fori_loop(start, end, chunk_body, jnp.int32(-1))


def _proj_body(x_ref, w_ref, b_ref, o_ref):
    o_ref[0] = (
        jnp.dot(w_ref[...], x_ref[0], preferred_element_type=jnp.float32)
        + b_ref[...]
    )


def _mlp_body(x_ref, w1_ref, b1_ref, w2_ref, b2_ref, o_ref):
    x = x_ref[0]
    h = jnp.dot(w1_ref[...], x, preferred_element_type=jnp.float32) + b1_ref[...]
    h = 0.5 * h * (1.0 + lax.erf(h * (2.0 ** -0.5)))
    o_ref[0] = (
        jnp.dot(w2_ref[...], h, preferred_element_type=jnp.float32)
        + b2_ref[...]
        + x
    )


def _proj(x, w, b, n):
    return pl.pallas_call(
        _proj_body,
        grid=(n,),
        in_specs=[
            pl.BlockSpec((1, _C, _AREA), lambda i: (i, 0, 0)),
            pl.BlockSpec((_C, _C), lambda i: (0, 0)),
            pl.BlockSpec((_C, 1), lambda i: (0, 0)),
        ],
        out_specs=pl.BlockSpec((1, _C, _AREA), lambda i: (i, 0, 0)),
        out_shape=jax.ShapeDtypeStruct((n, _C, _AREA), jnp.float32),
    )(x, w, b[:, None])


def kernel(q, k, v, offset, Wq, bq, Wk, bk, Wv, bv, W1, b1, W2, b2):
    qp = _proj(q.reshape(_B, _C, _AREA), Wq, bq, _B)
    kp = _proj(k.reshape(_B * _CLIP, _C, _AREA), Wk, bk, _B * _CLIP)
    vp = _proj(v.reshape(_B * _CLIP, _C, _AREA), Wv, bv, _B * _CLIP)

    pix = jnp.arange(_AREA, dtype=jnp.float32)
    kr = jnp.arange(_K2, dtype=jnp.float32)[:, None]
    gy = (pix // _W)[None, :] + (jnp.floor(kr / 3.0) - 1.0)
    gx = (pix % _W)[None, :] + (kr % 3.0 - 1.0)
    grid = jnp.stack([gy, gx], axis=1)

    aout = _sc_attn(
        kp.reshape(_B * _CLIP * _C * _AREA),
        vp.reshape(_B * _CLIP * _C * _AREA),
        qp.reshape(_B, _G, _CG, _AREA),
        offset.reshape(_B, _CLIP, _G, _K2, 2, _AREA),
        grid,
    ).reshape(_B, _C, _AREA)

    y = pl.pallas_call(
        _mlp_body,
        grid=(_B, 3),
        in_specs=[
            pl.BlockSpec((1, _C, _AREA // 3), lambda i, j: (i, 0, j)),
            pl.BlockSpec((2 * _C, _C), lambda i, j: (0, 0)),
            pl.BlockSpec((2 * _C, 1), lambda i, j: (0, 0)),
            pl.BlockSpec((_C, 2 * _C), lambda i, j: (0, 0)),
            pl.BlockSpec((_C, 1), lambda i, j: (0, 0)),
        ],
        out_specs=pl.BlockSpec((1, _C, _AREA // 3), lambda i, j: (i, 0, j)),
        out_shape=jax.ShapeDtypeStruct((_B, _C, _AREA), jnp.float32),
    )(aout, W1, b1[:, None], W2, b2[:, None])

    return y.reshape(_B, 1, _C, _H, _W)


# speedup vs baseline: 1.0229x; 1.0229x over previous
import functools

import jax
import jax.numpy as jnp
from jax import lax
from jax.experimental import pallas as pl
from jax.experimental.pallas import tpu as pltpu
from jax.experimental.pallas import tpu_sc as plsc

_B = 2
_CLIP = 2
_C = 144
_H = 48
_W = 48
_AREA = _H * _W
_G = 12
_CG = _C // _G
_K2 = 9
_NS = _CLIP * _K2
_BG = _B * _G
_NCH = 9
_CHUNK = _AREA // _NCH
_NBLK = _CHUNK // 16
_SCALE = float(_CG) ** -0.5

_mesh = plsc.VectorSubcoreMesh(core_axis_name="c", subcore_axis_name="s")


def _const16(v):
    return jnp.full((16,), v, jnp.int32)


@functools.partial(
    pl.kernel,
    out_type=jax.ShapeDtypeStruct((_B, _G, _CG, _AREA), jnp.float32),
    mesh=_mesh,
    scratch_types=[
        pltpu.VMEM((4 * _CG * _AREA,), jnp.float32),
        pltpu.VMEM((_CLIP, _K2, 2, _CHUNK), jnp.float32),
        pltpu.VMEM((_CG, _CHUNK), jnp.float32),
        pltpu.VMEM((_CG, _CHUNK), jnp.float32),
        pltpu.VMEM((_NS * 4 * 16,), jnp.float32),
        pltpu.VMEM((_NS * 4 * 16,), jnp.int32),
        pltpu.VMEM((_NS * 16,), jnp.float32),
    ],
    compiler_params=pltpu.CompilerParams(needs_layout_passes=False),
)
def _sc_attn(kp_hbm, vp_hbm, qp_hbm, off_hbm, out_hbm,
             kvt, offv, qv, outv, wbuf, ibuf, lbuf):
    wid = lax.axis_index("s") * 2 + lax.axis_index("c")
    ncg = _BG * _NCH
    start = (wid * ncg) // 32
    end = ((wid + 1) * ncg) // 32

    def chunk_body(cg, prev_bg):
        bg = cg // _NCH
        ch = cg - bg * _NCH
        b = bg // _G
        g = bg - b * _G

        clw = _CG * _AREA
        @pl.when(bg != prev_bg)
        def _load_table():
            for clip in range(_CLIP):
                base = (b * 2 + clip) * (_C * _AREA) + g * clw
                pltpu.sync_copy(
                    kp_hbm.at[pl.ds(base, clw)], kvt.at[pl.ds(clip * clw, clw)]
                )
                pltpu.sync_copy(
                    vp_hbm.at[pl.ds(base, clw)],
                    kvt.at[pl.ds((2 + clip) * clw, clw)],
                )

        for clip in range(_CLIP):
            pltpu.sync_copy(
                off_hbm.at[b, clip, g, :, :, pl.ds(ch * _CHUNK, _CHUNK)],
                offv.at[clip],
            )
        pltpu.sync_copy(qp_hbm.at[b, g, :, pl.ds(ch * _CHUNK, _CHUNK)], qv)

        def blk_body(blk, carry1):
            p0 = blk * 16
            pix = ch * _CHUNK + p0 + lax.iota(jnp.int32, 16)
            hv = pix // _W
            wv = pix - hv * _W
            qs = [qv[c, pl.ds(p0, 16)] for c in range(_CG)]

            def make_s1(clip):
                def s1(kpos, carry2):
                    s = kpos + clip * _K2
                    ki = kpos // 3
                    kj = kpos - ki * 3
                    sy = (hv + (ki - 1)).astype(jnp.float32) + offv[clip, kpos, 0, pl.ds(p0, 16)]
                    sx = (wv + (kj - 1)).astype(jnp.float32) + offv[clip, kpos, 1, pl.ds(p0, 16)]
                    ty = sy.astype(jnp.int32)
                    y0 = jnp.where(sy < ty.astype(jnp.float32), ty - 1, ty)
                    tx = sx.astype(jnp.int32)
                    x0 = jnp.where(sx < tx.astype(jnp.float32), tx - 1, tx)
                    gs = []
                    ws = []
                    for t, (dy, dx) in enumerate(((0, 0), (0, 1), (1, 0), (1, 1))):
                        yi = y0 + dy
                        xi = x0 + dx
                        wy = 1.0 - jnp.abs(sy - yi.astype(jnp.float32))
                        wx = 1.0 - jnp.abs(sx - xi.astype(jnp.float32))
                        ok = (yi >= 0) & (yi <= _H - 1) & (xi >= 0) & (xi <= _W - 1)
                        wgt = jnp.where(ok, wy * wx, 0.0)
                        yc = jnp.clip(yi, 0, _H - 1)
                        xc = jnp.clip(xi, 0, _W - 1)
                        gi = yc * _W + xc
                        wbuf[pl.ds((s * 4 + t) * 16, 16)] = wgt
                        ibuf[pl.ds((s * 4 + t) * 16, 16)] = gi
                        gs.append(gi)
                        ws.append(wgt)
                    logit = jnp.zeros((16,), jnp.float32)
                    for c in range(_CG):
                        o = clip * (_CG * _AREA) + c * _AREA
                        kc = ws[0] * plsc.load_gather(kvt, [gs[0] + o])
                        for t in range(1, 4):
                            kc = kc + ws[t] * plsc.load_gather(kvt, [gs[t] + o])
                        logit = logit + qs[c] * kc
                    lbuf[pl.ds(s * 16, 16)] = logit * _SCALE
                    return carry2

                return s1

            lax.fori_loop(0, _K2, make_s1(0), 0)
            lax.fori_loop(0, _K2, make_s1(1), 0)

            def smax(s, m):
                return jnp.maximum(m, lbuf[pl.ds(s * 16, 16)])

            m = lax.fori_loop(1, _NS, smax, lbuf[pl.ds(0, 16)])

            def sexp(s, den):
                p = jnp.exp(lbuf[pl.ds(s * 16, 16)] - m)
                lbuf[pl.ds(s * 16, 16)] = p
                return den + p

            den = lax.fori_loop(0, _NS, sexp, jnp.zeros((16,), jnp.float32))
            rden = 1.0 / den

            def make_s2(clip):
                def s2(kpos, acc):
                    s = kpos + clip * _K2
                    a = lbuf[pl.ds(s * 16, 16)] * rden
                    accl = list(acc)
                    for t in range(4):
                        aw = a * wbuf[pl.ds((s * 4 + t) * 16, 16)]
                        gi = ibuf[pl.ds((s * 4 + t) * 16, 16)]
                        for c in range(_CG):
                            o = (2 + clip) * (_CG * _AREA) + c * _AREA
                            accl[c] = accl[c] + aw * plsc.load_gather(
                                kvt, [gi + o]
                            )
                    return tuple(accl)

                return s2

            zero = jnp.zeros((16,), jnp.float32)
            acc = lax.fori_loop(0, _K2, make_s2(0), (zero,) * _CG)
            acc = lax.fori_loop(0, _K2, make_s2(1), acc)
            for c in range(_CG):
                outv[c, pl.ds(p0, 16)] = acc[c]
            return carry1

        lax.fori_loop(0, _NBLK, blk_body, 0)
        pltpu.sync_copy(
            outv, out_hbm.at[b, g, :, pl.ds(ch * _CHUNK, _CHUNK)]
        )
        return bg

    lax.fori_loop(start, end, chunk_body, jnp.int32(-1))


def _proj_body(x_ref, w_ref, b_ref, o_ref):
    o_ref[0] = (
        jnp.dot(w_ref[...], x_ref[0], preferred_element_type=jnp.float32)
        + b_ref[...]
    )


def _mlp_body(x_ref, w1_ref, b1_ref, w2_ref, b2_ref, o_ref):
    x = x_ref[0]
    h = jnp.dot(w1_ref[...], x, preferred_element_type=jnp.float32) + b1_ref[...]
    h = 0.5 * h * (1.0 + lax.erf(h * (2.0 ** -0.5)))
    o_ref[0] = (
        jnp.dot(w2_ref[...], h, preferred_element_type=jnp.float32)
        + b2_ref[...]
        + x
    )


def _proj(x, w, b, n):
    return pl.pallas_call(
        _proj_body,
        grid=(n,),
        in_specs=[
            pl.BlockSpec((1, _C, _AREA), lambda i: (i, 0, 0)),
            pl.BlockSpec((_C, _C), lambda i: (0, 0)),
            pl.BlockSpec((_C, 1), lambda i: (0, 0)),
        ],
        out_specs=pl.BlockSpec((1, _C, _AREA), lambda i: (i, 0, 0)),
        out_shape=jax.ShapeDtypeStruct((n, _C, _AREA), jnp.float32),
    )(x, w, b[:, None])


def kernel(q, k, v, offset, Wq, bq, Wk, bk, Wv, bv, W1, b1, W2, b2):
    qp = _proj(q.reshape(_B, _C, _AREA), Wq, bq, _B)
    kp = _proj(k.reshape(_B * _CLIP, _C, _AREA), Wk, bk, _B * _CLIP)
    vp = _proj(v.reshape(_B * _CLIP, _C, _AREA), Wv, bv, _B * _CLIP)

    aout = _sc_attn(
        kp.reshape(_B * _CLIP * _C * _AREA),
        vp.reshape(_B * _CLIP * _C * _AREA),
        qp.reshape(_B, _G, _CG, _AREA),
        offset.reshape(_B, _CLIP, _G, _K2, 2, _AREA),
    ).reshape(_B, _C, _AREA)

    y = pl.pallas_call(
        _mlp_body,
        grid=(_B, 3),
        in_specs=[
            pl.BlockSpec((1, _C, _AREA // 3), lambda i, j: (i, 0, j)),
            pl.BlockSpec((2 * _C, _C), lambda i, j: (0, 0)),
            pl.BlockSpec((2 * _C, 1), lambda i, j: (0, 0)),
            pl.BlockSpec((_C, 2 * _C), lambda i, j: (0, 0)),
            pl.BlockSpec((_C, 1), lambda i, j: (0, 0)),
        ],
        out_specs=pl.BlockSpec((1, _C, _AREA // 3), lambda i, j: (i, 0, j)),
        out_shape=jax.ShapeDtypeStruct((_B, _C, _AREA), jnp.float32),
    )(aout, W1, b1[:, None], W2, b2[:, None])

    return y.reshape(_B, 1, _C, _H, _W)
